# baseline (speedup 1.0000x reference)
import functools

import jax
import jax.numpy as jnp
from jax import lax
from jax.experimental import pallas as pl
from jax.experimental.pallas import tpu as pltpu
from jax.experimental.pallas import tpu_sc as plsc

_N = 10000
_E = 160000
_DIN = 256
_H = 512
_C = 16
_G = 64

_CW = 128
_NCHUNK = _H // _CW
_BLK = 64
_NBLK = 160
_RING = 4
_EPT = _NBLK * _BLK
_EPAD = 16 * _EPT
_ROWS = 10240
_DUMMY = 10200
_MB = 1000
_GRID = _N // _MB
_BF = jnp.bfloat16


def _make_conv_sc(with_cnt: bool):
    mesh = plsc.VectorSubcoreMesh(core_axis_name="c", subcore_axis_name="s")
    out_type = [jax.ShapeDtypeStruct((_NCHUNK, _ROWS, _CW), jnp.float32)]
    if with_cnt:
        out_type.append(
            jax.ShapeDtypeStruct((2, _ROWS, _CW), jnp.float32))
    scratch = [
        pltpu.VMEM((_RING, _BLK), jnp.int32),
        pltpu.VMEM((_RING, _BLK), jnp.int32),
        pltpu.VMEM((_RING, _BLK, _CW), jnp.float32),
        pltpu.VMEM((16, _CW), jnp.float32),
        pltpu.VMEM_SHARED((_ROWS, _CW), jnp.float32),
    ] + [pltpu.SemaphoreType.DMA] * (2 * _RING)
    rpt = _ROWS // 16

    @functools.partial(pl.kernel, out_type=out_type, mesh=mesh,
                       scratch_types=scratch)
    def conv(*refs):
        if with_cnt:
            (h4_ref, src4_ref, dst_ref, zeros_hbm, ones_hbm, agg_ref,
             cnt_ref, idx_v, dst_v, msg_v, zero_v, acc_sh, *sems) = refs
        else:
            (h4_ref, src4_ref, dst_ref, zeros_hbm, ones_hbm, agg_ref,
             idx_v, dst_v, msg_v, zero_v, acc_sh, *sems) = refs
        gsem = sems[:_RING]
        ssem = sems[_RING:]
        cid = lax.axis_index("c")
        sid = lax.axis_index("s")

        pltpu.sync_copy(zeros_hbm, zero_v)

        def zero_acc():
            hs = [pltpu.async_copy(
                zero_v, acc_sh.at[pl.ds(sid * rpt + j * 16, 16)],
                gsem[j % _RING]) for j in range(rpt // 16)]
            for h in hs:
                h.wait()

        def drain_scatter(p):
            pltpu.make_async_copy(
                msg_v.at[p], acc_sh.at[dst_v.at[p]], ssem[p]).wait()

        if with_cnt:
            pltpu.sync_copy(ones_hbm, msg_v.at[0])
            zero_acc()
            plsc.subcore_barrier()
            lo = lax.select(cid == 0, 0, _NBLK // 2)

            def cbody(i, carry):
                dh = []
                for p in range(_RING):
                    @pl.when(i > 0)
                    def _():
                        drain_scatter(p)
                    b = lo + _RING * i + p
                    dh.append(pltpu.async_copy(
                        dst_ref.at[sid, b], dst_v.at[p], gsem[p]))
                for p in range(_RING):
                    dh[p].wait()
                    pltpu.async_copy(
                        msg_v.at[0], acc_sh.at[dst_v.at[p]], ssem[p],
                        add=True)
                return carry
            lax.fori_loop(0, _NBLK // 2 // _RING, cbody, 0)
            for p in range(_RING):
                drain_scatter(p)
            plsc.subcore_barrier()
            pltpu.sync_copy(acc_sh.at[pl.ds(sid * rpt, rpt)],
                            cnt_ref.at[cid, pl.ds(sid * rpt, rpt)])
            plsc.subcore_barrier()

        for kk in range(2):
            ck = 2 * cid + kk
            zero_acc()
            plsc.subcore_barrier()

            def body(i, carry):
                ih, gh, dh = [], [], []
                for p in range(_RING):
                    @pl.when(i > 0)
                    def _():
                        drain_scatter(p)
                    b = _RING * i + p
                    ih.append(pltpu.async_copy(
                        src4_ref.at[ck, sid, b], idx_v.at[p], gsem[p]))
                    dh.append(pltpu.async_copy(
                        dst_ref.at[sid, b], dst_v.at[p], ssem[p]))
                for p in range(_RING):
                    ih[p].wait()
                    gh.append(pltpu.async_copy(
                        h4_ref.at[idx_v.at[p]], msg_v.at[p], gsem[p]))
                for p in range(_RING):
                    dh[p].wait()
                    gh[p].wait()
                    pltpu.async_copy(
                        msg_v.at[p], acc_sh.at[dst_v.at[p]], ssem[p],
                        add=True)
                return carry
            lax.fori_loop(0, _NBLK // _RING, body, 0)
            for p in range(_RING):
                drain_scatter(p)
            plsc.subcore_barrier()

            pltpu.sync_copy(acc_sh.at[pl.ds(sid * rpt, rpt)],
                            agg_ref.at[ck, pl.ds(sid * rpt, rpt)])
            plsc.subcore_barrier()

    return conv


_conv_sc_cnt = _make_conv_sc(True)
_conv_sc = _make_conv_sc(False)


def _bdot(a, b):
    return jnp.dot(a.astype(_BF), b.astype(_BF),
                   preferred_element_type=jnp.float32)


def _emb_body(x_ref, w1_ref, b1_ref, w2_ref, b2_ref, out_ref):
    h = _bdot(x_ref[...], w1_ref[...]) + b1_ref[...]
    h = jnp.maximum(h, 0.0)
    out_ref[...] = _bdot(h, w2_ref[...]) + b2_ref[...]


def _emb_mlp(x, w1, b1, w2, b2):
    return pl.pallas_call(
        _emb_body,
        grid=(_GRID,),
        in_specs=[
            pl.BlockSpec((_MB, _DIN), lambda i: (i, 0)),
            pl.BlockSpec((_DIN, _H), lambda i: (0, 0)),
            pl.BlockSpec((1, _H), lambda i: (0, 0)),
            pl.BlockSpec((_H, _H), lambda i: (0, 0)),
            pl.BlockSpec((1, _H), lambda i: (0, 0)),
        ],
        out_specs=pl.BlockSpec((_MB, _H), lambda i: (i, 0)),
        out_shape=jax.ShapeDtypeStruct((_N, _H), jnp.float32),
    )(x, w1, b1, w2, b2)


def _conv_tail(agg_ref, rc, h_ref, wrel_ref, brel_ref, wroot_ref,
               wa_ref, ba_ref, wb_ref, bb_ref):
    t = _bdot(h_ref[...], wroot_ref[...])
    for k in range(_NCHUNK):
        mean_k = agg_ref[k] * rc
        t += _bdot(mean_k, wrel_ref[pl.ds(k * _CW, _CW), :])
    t += brel_ref[...]
    u = jnp.maximum(_bdot(t, wa_ref[...]) + ba_ref[...], 0.0)
    return jnp.maximum(_bdot(u, wb_ref[...]) + bb_ref[...], 0.0)


def _counts(cnt_ref):
    c = cnt_ref[0, :, :1] + cnt_ref[1, :, :1]
    return 1.0 / jnp.maximum(c, 1.0)


def _mid_body(agg_ref, cnt_ref, h_ref, wrel_ref, brel_ref, wroot_ref,
              wa_ref, ba_ref, wb_ref, bb_ref, out_ref):
    out_ref[...] = _conv_tail(agg_ref, _counts(cnt_ref), h_ref, wrel_ref,
                              brel_ref, wroot_ref, wa_ref, ba_ref, wb_ref,
                              bb_ref)


_W_SPECS = [
    pl.BlockSpec((_H, _H), lambda i: (0, 0)),
    pl.BlockSpec((1, _H), lambda i: (0, 0)),
    pl.BlockSpec((_H, _H), lambda i: (0, 0)),
    pl.BlockSpec((_H, _H), lambda i: (0, 0)),
    pl.BlockSpec((1, _H), lambda i: (0, 0)),
    pl.BlockSpec((_H, _H), lambda i: (0, 0)),
    pl.BlockSpec((1, _H), lambda i: (0, 0)),
]

_AGG_SPEC = pl.BlockSpec((_NCHUNK, _MB, _CW), lambda i: (0, i, 0))
_CNT_SPEC = pl.BlockSpec((2, _MB, _CW), lambda i: (0, i, 0))


def _mid_stage(agg, cnt, h, wrel, brel, wroot, wa, ba, wb, bb):
    return pl.pallas_call(
        _mid_body,
        grid=(_GRID,),
        in_specs=[
            _AGG_SPEC,
            _CNT_SPEC,
            pl.BlockSpec((_MB, _H), lambda i: (i, 0)),
        ] + _W_SPECS,
        out_specs=pl.BlockSpec((_MB, _H), lambda i: (i, 0)),
        out_shape=jax.ShapeDtypeStruct((_N, _H), jnp.float32),
    )(agg, cnt, h, wrel, brel, wroot, wa, ba, wb, bb)


def _final_body(agg_ref, cnt_ref, h_ref, batch_ref, wrel_ref, brel_ref,
                wroot_ref, wa_ref, ba_ref, wb_ref, bb_ref, wcls_ref, bcls_ref,
                out_ref, sums_sc, gcnt_sc):
    i = pl.program_id(0)

    @pl.when(i == 0)
    def _():
        sums_sc[...] = jnp.zeros_like(sums_sc)
        gcnt_sc[...] = jnp.zeros_like(gcnt_sc)

    h2 = _conv_tail(agg_ref, _counts(cnt_ref), h_ref, wrel_ref, brel_ref,
                    wroot_ref, wa_ref, ba_ref, wb_ref, bb_ref)
    onehot = (batch_ref[...] ==
              lax.broadcasted_iota(jnp.int32, (1, _G), 1)).astype(jnp.float32)
    sums_sc[...] += lax.dot_general(onehot, h2, (((0,), (0,)), ((), ())),
                                    preferred_element_type=jnp.float32)
    gcnt_sc[...] += lax.dot_general(onehot, jnp.ones((_MB, 128), jnp.float32),
                                    (((0,), (0,)), ((), ())),
                                    preferred_element_type=jnp.float32)

    @pl.when(i == _GRID - 1)
    def _():
        pooled = sums_sc[...] * (1.0 / jnp.maximum(gcnt_sc[...][:, :1], 1.0))
        out_ref[...] = jnp.dot(pooled, wcls_ref[...],
                               preferred_element_type=jnp.float32) + bcls_ref[...]


def _final_stage(agg, cnt, h, batch2, wrel, brel, wroot, wa, ba, wb, bb,
                 wcls, bcls):
    return pl.pallas_call(
        _final_body,
        grid=(_GRID,),
        in_specs=[
            _AGG_SPEC,
            _CNT_SPEC,
            pl.BlockSpec((_MB, _H), lambda i: (i, 0)),
            pl.BlockSpec((_MB, 1), lambda i: (i, 0)),
        ] + _W_SPECS + [
            pl.BlockSpec((_H, _C), lambda i: (0, 0)),
            pl.BlockSpec((1, _C), lambda i: (0, 0)),
        ],
        out_specs=pl.BlockSpec((_G, _C), lambda i: (0, 0)),
        out_shape=jax.ShapeDtypeStruct((_G, _C), jnp.float32),
        scratch_shapes=[
            pltpu.VMEM((_G, _H), jnp.float32),
            pltpu.VMEM((_G, 128), jnp.float32),
        ],
        compiler_params=pltpu.CompilerParams(
            dimension_semantics=("arbitrary",)),
    )(agg, cnt, h, batch2, wrel, brel, wroot, wa, ba, wb, bb, wcls, bcls)


def kernel(x, edge_index, batch, W_emb1, b_emb1, W_emb2, b_emb2,
           W_rel1, b_rel1, W_root1, W_p1a, b_p1a, W_p1b, b_p1b,
           W_rel2, b_rel2, W_root2, W_p2a, b_p2a, W_p2b, b_p2b,
           W_cls, b_cls):
    src = edge_index[0]
    dst = edge_index[1]
    srcp = jnp.pad(src, (0, _EPAD - _E)).reshape(16, _NBLK, _BLK)
    src4 = (srcp[None] * 4 +
            jnp.arange(4, dtype=jnp.int32).reshape(4, 1, 1, 1))
    dstp = jnp.pad(dst, (0, _EPAD - _E),
                   constant_values=_DUMMY).reshape(16, _NBLK, _BLK)

    r1 = lambda v: v.reshape(1, -1)
    h0 = _emb_mlp(x, W_emb1, r1(b_emb1), W_emb2, r1(b_emb2))

    zeros_p = jnp.zeros((16, _CW), jnp.float32)
    ones_p = jnp.ones((_BLK, _CW), jnp.float32)
    agg1, cnt = _conv_sc_cnt(h0.reshape(_N * _NCHUNK, _CW), src4, dstp,
                             zeros_p, ones_p)

    h1 = _mid_stage(agg1, cnt, h0, W_rel1, r1(b_rel1), W_root1,
                    W_p1a, r1(b_p1a), W_p1b, r1(b_p1b))

    (agg2,) = _conv_sc(h1.reshape(_N * _NCHUNK, _CW), src4, dstp,
                       zeros_p, ones_p)

    return _final_stage(agg2, cnt, h1, batch.reshape(_N, 1), W_rel2,
                        r1(b_rel2), W_root2, W_p2a, r1(b_p2a), W_p2b,
                        r1(b_p2b), W_cls, r1(b_cls))


# speedup vs baseline: 1.1020x; 1.1020x over previous
import functools

import jax
import jax.numpy as jnp
from jax import lax
from jax.experimental import pallas as pl
from jax.experimental.pallas import tpu as pltpu
from jax.experimental.pallas import tpu_sc as plsc

_N = 10000
_E = 160000
_DIN = 256
_H = 512
_C = 16
_G = 64

_CW = 128
_NCHUNK = _H // _CW
_BLK = 128
_NBLK = 80
_EPT = _NBLK * _BLK
_EPAD = 16 * _EPT
_ROWS = 10240
_DUMMY = 10200
_MB = 1000
_GRID = _N // _MB
_BF = jnp.bfloat16


_MESH = plsc.VectorSubcoreMesh(core_axis_name="c", subcore_axis_name="s")
_RPT = _ROWS // 16


def _acc_scratch():
    return [
        pltpu.VMEM((_NBLK, _BLK), jnp.int32),
        pltpu.VMEM((2, _BLK), jnp.int32),
        pltpu.VMEM((2, _BLK, _CW), jnp.float32),
        pltpu.VMEM((16, _CW), jnp.float32),
        pltpu.VMEM_SHARED((_ROWS, _CW), jnp.float32),
    ] + [pltpu.SemaphoreType.DMA] * 4


def _zero_acc(zero_v, acc_sh, sid, gsem):
    hs = [pltpu.async_copy(
        zero_v, acc_sh.at[pl.ds(sid * _RPT + j * 16, 16)],
        gsem[j % 2]) for j in range(_RPT // 16)]
    for h in hs:
        h.wait()


@functools.partial(pl.kernel,
                   out_type=[jax.ShapeDtypeStruct((2, _ROWS, _CW),
                                                  jnp.float32)],
                   mesh=_MESH, scratch_types=_acc_scratch())
def _cnt_sc(dst_ref, zeros_hbm, ones_hbm, cnt_ref,
            idx_v, dst_v, msg_v, zero_v, acc_sh, g0, g1, s0, s1):
    gsem = (g0, g1)
    ssem = (s0, s1)
    cid = lax.axis_index("c")
    sid = lax.axis_index("s")

    pltpu.sync_copy(zeros_hbm, zero_v)
    pltpu.sync_copy(ones_hbm, msg_v.at[0])

    def drain_scatter(p):
        pltpu.make_async_copy(
            msg_v.at[0], acc_sh.at[dst_v.at[p]], ssem[p]).wait()

    _zero_acc(zero_v, acc_sh, sid, gsem)
    plsc.subcore_barrier()
    lo = lax.select(cid == 0, 0, _NBLK // 2)

    def cbody(i, carry):
        dh = []
        for p in range(2):
            @pl.when(i > 0)
            def _():
                drain_scatter(p)
            b = lo + 2 * i + p
            dh.append(pltpu.async_copy(
                dst_ref.at[sid, b], dst_v.at[p], gsem[p]))
        for p in range(2):
            dh[p].wait()
            pltpu.async_copy(
                msg_v.at[0], acc_sh.at[dst_v.at[p]], ssem[p], add=True)
        return carry
    lax.fori_loop(0, _NBLK // 4, cbody, 0)
    for p in range(2):
        drain_scatter(p)
    plsc.subcore_barrier()
    pltpu.sync_copy(acc_sh.at[pl.ds(sid * _RPT, _RPT)],
                    cnt_ref.at[cid, pl.ds(sid * _RPT, _RPT)])
    plsc.subcore_barrier()


@functools.partial(pl.kernel,
                   out_type=[jax.ShapeDtypeStruct((_NCHUNK, _ROWS, _CW),
                                                  jnp.float32)],
                   mesh=_MESH, scratch_types=_acc_scratch())
def _conv_sc(h4_ref, src4_ref, dst_ref, zeros_hbm, ones_hbm, agg_ref,
             idx_v, dst_v, msg_v, zero_v, acc_sh, g0, g1, s0, s1):
    gsem = (g0, g1)
    ssem = (s0, s1)
    cid = lax.axis_index("c")
    sid = lax.axis_index("s")

    pltpu.sync_copy(zeros_hbm, zero_v)

    def drain_scatter(p):
        pltpu.make_async_copy(
            msg_v.at[p], acc_sh.at[dst_v.at[p]], ssem[p]).wait()

    for kk in range(2):
        ck = 2 * cid + kk
        _zero_acc(zero_v, acc_sh, sid, gsem)
        pltpu.sync_copy(src4_ref.at[ck, sid], idx_v)
        plsc.subcore_barrier()

        def body(i, carry):
            gh, dh = [], []
            for p in range(2):
                @pl.when(i > 0)
                def _():
                    drain_scatter(p)
                b = 2 * i + p
                gh.append(pltpu.async_copy(
                    h4_ref.at[idx_v.at[b]], msg_v.at[p], gsem[p]))
                dh.append(pltpu.async_copy(
                    dst_ref.at[sid, b], dst_v.at[p], ssem[p]))
            for p in range(2):
                dh[p].wait()
                gh[p].wait()
                pltpu.async_copy(
                    msg_v.at[p], acc_sh.at[dst_v.at[p]], ssem[p], add=True)
            return carry
        lax.fori_loop(0, _NBLK // 2, body, 0)
        for p in range(2):
            drain_scatter(p)
        plsc.subcore_barrier()

        pltpu.sync_copy(acc_sh.at[pl.ds(sid * _RPT, _RPT)],
                        agg_ref.at[ck, pl.ds(sid * _RPT, _RPT)])
        plsc.subcore_barrier()


def _bdot(a, b):
    return jnp.dot(a.astype(_BF), b.astype(_BF),
                   preferred_element_type=jnp.float32)


def _emb_body(x_ref, w1_ref, b1_ref, w2_ref, b2_ref, out_ref):
    h = _bdot(x_ref[...], w1_ref[...]) + b1_ref[...]
    h = jnp.maximum(h, 0.0)
    out_ref[...] = _bdot(h, w2_ref[...]) + b2_ref[...]


def _emb_mlp(x, w1, b1, w2, b2):
    return pl.pallas_call(
        _emb_body,
        grid=(_GRID,),
        in_specs=[
            pl.BlockSpec((_MB, _DIN), lambda i: (i, 0)),
            pl.BlockSpec((_DIN, _H), lambda i: (0, 0)),
            pl.BlockSpec((1, _H), lambda i: (0, 0)),
            pl.BlockSpec((_H, _H), lambda i: (0, 0)),
            pl.BlockSpec((1, _H), lambda i: (0, 0)),
        ],
        out_specs=pl.BlockSpec((_MB, _H), lambda i: (i, 0)),
        out_shape=jax.ShapeDtypeStruct((_N, _H), jnp.float32),
    )(x, w1, b1, w2, b2)


def _conv_tail(agg_ref, rc, h_ref, wrel_ref, brel_ref, wroot_ref,
               wa_ref, ba_ref, wb_ref, bb_ref):
    t = _bdot(h_ref[...], wroot_ref[...])
    for k in range(_NCHUNK):
        mean_k = agg_ref[k] * rc
        t += _bdot(mean_k, wrel_ref[pl.ds(k * _CW, _CW), :])
    t += brel_ref[...]
    u = jnp.maximum(_bdot(t, wa_ref[...]) + ba_ref[...], 0.0)
    return jnp.maximum(_bdot(u, wb_ref[...]) + bb_ref[...], 0.0)


def _counts(cnt_ref):
    c = cnt_ref[0, :, :1] + cnt_ref[1, :, :1]
    return 1.0 / jnp.maximum(c, 1.0)


def _mid_body(agg_ref, cnt_ref, h_ref, wrel_ref, brel_ref, wroot_ref,
              wa_ref, ba_ref, wb_ref, bb_ref, out_ref):
    out_ref[...] = _conv_tail(agg_ref, _counts(cnt_ref), h_ref, wrel_ref,
                              brel_ref, wroot_ref, wa_ref, ba_ref, wb_ref,
                              bb_ref)


_W_SPECS = [
    pl.BlockSpec((_H, _H), lambda i: (0, 0)),
    pl.BlockSpec((1, _H), lambda i: (0, 0)),
    pl.BlockSpec((_H, _H), lambda i: (0, 0)),
    pl.BlockSpec((_H, _H), lambda i: (0, 0)),
    pl.BlockSpec((1, _H), lambda i: (0, 0)),
    pl.BlockSpec((_H, _H), lambda i: (0, 0)),
    pl.BlockSpec((1, _H), lambda i: (0, 0)),
]

_AGG_SPEC = pl.BlockSpec((_NCHUNK, _MB, _CW), lambda i: (0, i, 0))
_CNT_SPEC = pl.BlockSpec((2, _MB, _CW), lambda i: (0, i, 0))


def _mid_stage(agg, cnt, h, wrel, brel, wroot, wa, ba, wb, bb):
    return pl.pallas_call(
        _mid_body,
        grid=(_GRID,),
        in_specs=[
            _AGG_SPEC,
            _CNT_SPEC,
            pl.BlockSpec((_MB, _H), lambda i: (i, 0)),
        ] + _W_SPECS,
        out_specs=pl.BlockSpec((_MB, _H), lambda i: (i, 0)),
        out_shape=jax.ShapeDtypeStruct((_N, _H), jnp.float32),
    )(agg, cnt, h, wrel, brel, wroot, wa, ba, wb, bb)


def _final_body(agg_ref, cnt_ref, h_ref, batch_ref, wrel_ref, brel_ref,
                wroot_ref, wa_ref, ba_ref, wb_ref, bb_ref, wcls_ref, bcls_ref,
                out_ref, sums_sc, gcnt_sc):
    i = pl.program_id(0)

    @pl.when(i == 0)
    def _():
        sums_sc[...] = jnp.zeros_like(sums_sc)
        gcnt_sc[...] = jnp.zeros_like(gcnt_sc)

    h2 = _conv_tail(agg_ref, _counts(cnt_ref), h_ref, wrel_ref, brel_ref,
                    wroot_ref, wa_ref, ba_ref, wb_ref, bb_ref)
    onehot = (batch_ref[...] ==
              lax.broadcasted_iota(jnp.int32, (1, _G), 1)).astype(jnp.float32)
    sums_sc[...] += lax.dot_general(onehot, h2, (((0,), (0,)), ((), ())),
                                    preferred_element_type=jnp.float32)
    gcnt_sc[...] += lax.dot_general(onehot, jnp.ones((_MB, 128), jnp.float32),
                                    (((0,), (0,)), ((), ())),
                                    preferred_element_type=jnp.float32)

    @pl.when(i == _GRID - 1)
    def _():
        pooled = sums_sc[...] * (1.0 / jnp.maximum(gcnt_sc[...][:, :1], 1.0))
        out_ref[...] = jnp.dot(pooled, wcls_ref[...],
                               preferred_element_type=jnp.float32) + bcls_ref[...]


def _final_stage(agg, cnt, h, batch2, wrel, brel, wroot, wa, ba, wb, bb,
                 wcls, bcls):
    return pl.pallas_call(
        _final_body,
        grid=(_GRID,),
        in_specs=[
            _AGG_SPEC,
            _CNT_SPEC,
            pl.BlockSpec((_MB, _H), lambda i: (i, 0)),
            pl.BlockSpec((_MB, 1), lambda i: (i, 0)),
        ] + _W_SPECS + [
            pl.BlockSpec((_H, _C), lambda i: (0, 0)),
            pl.BlockSpec((1, _C), lambda i: (0, 0)),
        ],
        out_specs=pl.BlockSpec((_G, _C), lambda i: (0, 0)),
        out_shape=jax.ShapeDtypeStruct((_G, _C), jnp.float32),
        scratch_shapes=[
            pltpu.VMEM((_G, _H), jnp.float32),
            pltpu.VMEM((_G, 128), jnp.float32),
        ],
        compiler_params=pltpu.CompilerParams(
            dimension_semantics=("arbitrary",)),
    )(agg, cnt, h, batch2, wrel, brel, wroot, wa, ba, wb, bb, wcls, bcls)


def kernel(x, edge_index, batch, W_emb1, b_emb1, W_emb2, b_emb2,
           W_rel1, b_rel1, W_root1, W_p1a, b_p1a, W_p1b, b_p1b,
           W_rel2, b_rel2, W_root2, W_p2a, b_p2a, W_p2b, b_p2b,
           W_cls, b_cls):
    src = edge_index[0]
    dst = edge_index[1]
    srcp = jnp.pad(src, (0, _EPAD - _E)).reshape(16, _NBLK, _BLK)
    src4 = (srcp[None] * 4 +
            jnp.arange(4, dtype=jnp.int32).reshape(4, 1, 1, 1))
    dstp = jnp.pad(dst, (0, _EPAD - _E),
                   constant_values=_DUMMY).reshape(16, _NBLK, _BLK)

    r1 = lambda v: v.reshape(1, -1)
    zeros_p = jnp.zeros((16, _CW), jnp.float32)
    ones_p = jnp.ones((_BLK, _CW), jnp.float32)
    h0 = _emb_mlp(x, W_emb1, r1(b_emb1), W_emb2, r1(b_emb2))
    (cnt,) = _cnt_sc(dstp, zeros_p, ones_p)
    (agg1,) = _conv_sc(h0.reshape(_N * _NCHUNK, _CW), src4, dstp,
                       zeros_p, ones_p)

    h1 = _mid_stage(agg1, cnt, h0, W_rel1, r1(b_rel1), W_root1,
                    W_p1a, r1(b_p1a), W_p1b, r1(b_p1b))

    (agg2,) = _conv_sc(h1.reshape(_N * _NCHUNK, _CW), src4, dstp,
                       zeros_p, ones_p)

    return _final_stage(agg2, cnt, h1, batch.reshape(_N, 1), W_rel2,
                        r1(b_rel2), W_root2, W_p2a, r1(b_p2a), W_p2b,
                        r1(b_p2b), W_cls, r1(b_cls))


# speedup vs baseline: 1.1089x; 1.0063x over previous
import functools

import jax
import jax.numpy as jnp
from jax import lax
from jax.experimental import pallas as pl
from jax.experimental.pallas import tpu as pltpu
from jax.experimental.pallas import tpu_sc as plsc

_N = 10000
_E = 160000
_DIN = 256
_H = 512
_C = 16
_G = 64

_CW = 128
_NCHUNK = _H // _CW
_BLK = 128
_NBLK = 80
_EPT = _NBLK * _BLK
_EPAD = 16 * _EPT
_ROWS = 10240
_DUMMY = 10200
_MB = 2000
_GRID = _N // _MB
_BF = jnp.bfloat16


def _make_conv_sc(with_cnt: bool):
    mesh = plsc.VectorSubcoreMesh(core_axis_name="c", subcore_axis_name="s")
    out_type = [jax.ShapeDtypeStruct((_NCHUNK, _ROWS, _CW), jnp.float32)]
    if with_cnt:
        out_type.append(
            jax.ShapeDtypeStruct((2, _ROWS, _CW), jnp.float32))
    scratch = [
        pltpu.VMEM((_NBLK, _BLK), jnp.int32),
        pltpu.VMEM((2, _BLK), jnp.int32),
        pltpu.VMEM((2, _BLK, _CW), jnp.float32),
        pltpu.VMEM((16, _CW), jnp.float32),
        pltpu.VMEM_SHARED((_ROWS, _CW), jnp.float32),
    ] + [pltpu.SemaphoreType.DMA] * 4
    rpt = _ROWS // 16

    @functools.partial(pl.kernel, out_type=out_type, mesh=mesh,
                       scratch_types=scratch)
    def conv(*refs):
        if with_cnt:
            (h4_ref, src4_ref, dst_ref, zeros_hbm, ones_hbm, agg_ref,
             cnt_ref, idx_v, dst_v, msg_v, zero_v, acc_sh,
             g0, g1, s0, s1) = refs
        else:
            (h4_ref, src4_ref, dst_ref, zeros_hbm, ones_hbm, agg_ref,
             idx_v, dst_v, msg_v, zero_v, acc_sh,
             g0, g1, s0, s1) = refs
        gsem = (g0, g1)
        ssem = (s0, s1)
        cid = lax.axis_index("c")
        sid = lax.axis_index("s")

        pltpu.sync_copy(zeros_hbm, zero_v)

        def zero_acc():
            hs = [pltpu.async_copy(
                zero_v, acc_sh.at[pl.ds(sid * rpt + j * 16, 16)],
                gsem[j % 2]) for j in range(rpt // 16)]
            for h in hs:
                h.wait()

        def drain_scatter(p):
            pltpu.make_async_copy(
                msg_v.at[p], acc_sh.at[dst_v.at[p]], ssem[p]).wait()

        if with_cnt:
            pltpu.sync_copy(ones_hbm, msg_v.at[0])
            zero_acc()
            plsc.subcore_barrier()
            lo = lax.select(cid == 0, 0, _NBLK // 2)

            def cbody(i, carry):
                dh = []
                for p in range(2):
                    @pl.when(i > 0)
                    def _():
                        drain_scatter(p)
                    b = lo + 2 * i + p
                    dh.append(pltpu.async_copy(
                        dst_ref.at[sid, b], dst_v.at[p], gsem[p]))
                for p in range(2):
                    dh[p].wait()
                    pltpu.async_copy(
                        msg_v.at[0], acc_sh.at[dst_v.at[p]], ssem[p],
                        add=True)
                return carry
            lax.fori_loop(0, _NBLK // 4, cbody, 0)
            for p in range(2):
                drain_scatter(p)
            plsc.subcore_barrier()
            pltpu.sync_copy(acc_sh.at[pl.ds(sid * rpt, rpt)],
                            cnt_ref.at[cid, pl.ds(sid * rpt, rpt)])
            plsc.subcore_barrier()

        for kk in range(2):
            ck = 2 * cid + kk
            zero_acc()
            pltpu.sync_copy(src4_ref.at[ck, sid], idx_v)
            plsc.subcore_barrier()

            def body(i, carry):
                gh, dh = [], []
                for p in range(2):
                    @pl.when(i > 0)
                    def _():
                        drain_scatter(p)
                    b = 2 * i + p
                    gh.append(pltpu.async_copy(
                        h4_ref.at[idx_v.at[b]], msg_v.at[p], gsem[p]))
                    dh.append(pltpu.async_copy(
                        dst_ref.at[sid, b], dst_v.at[p], ssem[p]))
                for p in range(2):
                    dh[p].wait()
                    gh[p].wait()
                    pltpu.async_copy(
                        msg_v.at[p], acc_sh.at[dst_v.at[p]], ssem[p],
                        add=True)
                return carry
            lax.fori_loop(0, _NBLK // 2, body, 0)
            for p in range(2):
                drain_scatter(p)
            plsc.subcore_barrier()

            pltpu.sync_copy(acc_sh.at[pl.ds(sid * rpt, rpt)],
                            agg_ref.at[ck, pl.ds(sid * rpt, rpt)])
            plsc.subcore_barrier()

    return conv


_conv_sc_cnt = _make_conv_sc(True)
_conv_sc = _make_conv_sc(False)


def _bdot(a, b):
    return jnp.dot(a.astype(_BF), b.astype(_BF),
                   preferred_element_type=jnp.float32)


def _emb_body(x_ref, w1_ref, b1_ref, w2_ref, b2_ref, out_ref):
    h = _bdot(x_ref[...], w1_ref[...]) + b1_ref[...]
    h = jnp.maximum(h, 0.0)
    out_ref[...] = _bdot(h, w2_ref[...]) + b2_ref[...]


def _emb_mlp(x, w1, b1, w2, b2):
    return pl.pallas_call(
        _emb_body,
        grid=(_GRID,),
        in_specs=[
            pl.BlockSpec((_MB, _DIN), lambda i: (i, 0)),
            pl.BlockSpec((_DIN, _H), lambda i: (0, 0)),
            pl.BlockSpec((1, _H), lambda i: (0, 0)),
            pl.BlockSpec((_H, _H), lambda i: (0, 0)),
            pl.BlockSpec((1, _H), lambda i: (0, 0)),
        ],
        out_specs=pl.BlockSpec((_MB, _H), lambda i: (i, 0)),
        out_shape=jax.ShapeDtypeStruct((_N, _H), jnp.float32),
    )(x, w1, b1, w2, b2)


def _conv_tail(agg_ref, rc, h_ref, wrel_ref, brel_ref, wroot_ref,
               wa_ref, ba_ref, wb_ref, bb_ref):
    t = _bdot(h_ref[...], wroot_ref[...])
    for k in range(_NCHUNK):
        mean_k = agg_ref[k] * rc
        t += _bdot(mean_k, wrel_ref[pl.ds(k * _CW, _CW), :])
    t += brel_ref[...]
    u = jnp.maximum(_bdot(t, wa_ref[...]) + ba_ref[...], 0.0)
    return jnp.maximum(_bdot(u, wb_ref[...]) + bb_ref[...], 0.0)


def _counts(cnt_ref):
    c = cnt_ref[0, :, :1] + cnt_ref[1, :, :1]
    return 1.0 / jnp.maximum(c, 1.0)


def _mid_body(agg_ref, cnt_ref, h_ref, wrel_ref, brel_ref, wroot_ref,
              wa_ref, ba_ref, wb_ref, bb_ref, out_ref):
    out_ref[...] = _conv_tail(agg_ref, _counts(cnt_ref), h_ref, wrel_ref,
                              brel_ref, wroot_ref, wa_ref, ba_ref, wb_ref,
                              bb_ref)


_W_SPECS = [
    pl.BlockSpec((_H, _H), lambda i: (0, 0)),
    pl.BlockSpec((1, _H), lambda i: (0, 0)),
    pl.BlockSpec((_H, _H), lambda i: (0, 0)),
    pl.BlockSpec((_H, _H), lambda i: (0, 0)),
    pl.BlockSpec((1, _H), lambda i: (0, 0)),
    pl.BlockSpec((_H, _H), lambda i: (0, 0)),
    pl.BlockSpec((1, _H), lambda i: (0, 0)),
]

_AGG_SPEC = pl.BlockSpec((_NCHUNK, _MB, _CW), lambda i: (0, i, 0))
_CNT_SPEC = pl.BlockSpec((2, _MB, _CW), lambda i: (0, i, 0))


def _mid_stage(agg, cnt, h, wrel, brel, wroot, wa, ba, wb, bb):
    return pl.pallas_call(
        _mid_body,
        grid=(_GRID,),
        in_specs=[
            _AGG_SPEC,
            _CNT_SPEC,
            pl.BlockSpec((_MB, _H), lambda i: (i, 0)),
        ] + _W_SPECS,
        out_specs=pl.BlockSpec((_MB, _H), lambda i: (i, 0)),
        out_shape=jax.ShapeDtypeStruct((_N, _H), jnp.float32),
    )(agg, cnt, h, wrel, brel, wroot, wa, ba, wb, bb)


def _final_body(agg_ref, cnt_ref, h_ref, batch_ref, wrel_ref, brel_ref,
                wroot_ref, wa_ref, ba_ref, wb_ref, bb_ref, wcls_ref, bcls_ref,
                out_ref, sums_sc, gcnt_sc):
    i = pl.program_id(0)

    @pl.when(i == 0)
    def _():
        sums_sc[...] = jnp.zeros_like(sums_sc)
        gcnt_sc[...] = jnp.zeros_like(gcnt_sc)

    h2 = _conv_tail(agg_ref, _counts(cnt_ref), h_ref, wrel_ref, brel_ref,
                    wroot_ref, wa_ref, ba_ref, wb_ref, bb_ref)
    onehot = (batch_ref[...] ==
              lax.broadcasted_iota(jnp.int32, (1, _G), 1)).astype(jnp.float32)
    sums_sc[...] += lax.dot_general(onehot, h2, (((0,), (0,)), ((), ())),
                                    preferred_element_type=jnp.float32)
    gcnt_sc[...] += lax.dot_general(onehot, jnp.ones((_MB, 128), jnp.float32),
                                    (((0,), (0,)), ((), ())),
                                    preferred_element_type=jnp.float32)

    @pl.when(i == _GRID - 1)
    def _():
        pooled = sums_sc[...] * (1.0 / jnp.maximum(gcnt_sc[...][:, :1], 1.0))
        out_ref[...] = jnp.dot(pooled, wcls_ref[...],
                               preferred_element_type=jnp.float32) + bcls_ref[...]


def _final_stage(agg, cnt, h, batch2, wrel, brel, wroot, wa, ba, wb, bb,
                 wcls, bcls):
    return pl.pallas_call(
        _final_body,
        grid=(_GRID,),
        in_specs=[
            _AGG_SPEC,
            _CNT_SPEC,
            pl.BlockSpec((_MB, _H), lambda i: (i, 0)),
            pl.BlockSpec((_MB, 1), lambda i: (i, 0)),
        ] + _W_SPECS + [
            pl.BlockSpec((_H, _C), lambda i: (0, 0)),
            pl.BlockSpec((1, _C), lambda i: (0, 0)),
        ],
        out_specs=pl.BlockSpec((_G, _C), lambda i: (0, 0)),
        out_shape=jax.ShapeDtypeStruct((_G, _C), jnp.float32),
        scratch_shapes=[
            pltpu.VMEM((_G, _H), jnp.float32),
            pltpu.VMEM((_G, 128), jnp.float32),
        ],
        compiler_params=pltpu.CompilerParams(
            dimension_semantics=("arbitrary",)),
    )(agg, cnt, h, batch2, wrel, brel, wroot, wa, ba, wb, bb, wcls, bcls)


def kernel(x, edge_index, batch, W_emb1, b_emb1, W_emb2, b_emb2,
           W_rel1, b_rel1, W_root1, W_p1a, b_p1a, W_p1b, b_p1b,
           W_rel2, b_rel2, W_root2, W_p2a, b_p2a, W_p2b, b_p2b,
           W_cls, b_cls):
    src = edge_index[0]
    dst = edge_index[1]
    srcp = jnp.pad(src, (0, _EPAD - _E)).reshape(16, _NBLK, _BLK)
    src4 = (srcp[None] * 4 +
            jnp.arange(4, dtype=jnp.int32).reshape(4, 1, 1, 1))
    dstp = jnp.pad(dst, (0, _EPAD - _E),
                   constant_values=_DUMMY).reshape(16, _NBLK, _BLK)

    r1 = lambda v: v.reshape(1, -1)
    h0 = _emb_mlp(x, W_emb1, r1(b_emb1), W_emb2, r1(b_emb2))

    zeros_p = jnp.zeros((16, _CW), jnp.float32)
    ones_p = jnp.ones((_BLK, _CW), jnp.float32)
    agg1, cnt = _conv_sc_cnt(h0.reshape(_N * _NCHUNK, _CW), src4, dstp,
                             zeros_p, ones_p)

    h1 = _mid_stage(agg1, cnt, h0, W_rel1, r1(b_rel1), W_root1,
                    W_p1a, r1(b_p1a), W_p1b, r1(b_p1b))

    (agg2,) = _conv_sc(h1.reshape(_N * _NCHUNK, _CW), src4, dstp,
                       zeros_p, ones_p)

    return _final_stage(agg2, cnt, h1, batch.reshape(_N, 1), W_rel2,
                        r1(b_rel2), W_root2, W_p2a, r1(b_p2a), W_p2b,
                        r1(b_p2b), W_cls, r1(b_cls))
